# Initial kernel scaffold; baseline (speedup 1.0000x reference)
import jax
import jax.numpy as jnp
from jax.experimental import pallas as pl


def kernel(x, edge_index, edge_weight, W1, b1, W2, b2, W3, b3):
    raise NotImplementedError("write your pallas kernel here")


# speedup vs baseline: 9.4420x; 9.4420x over previous
import functools

import jax
import jax.numpy as jnp
from jax import lax
from jax.experimental import pallas as pl
from jax.experimental.pallas import tpu as pltpu
from jax.experimental.pallas import tpu_sc as plsc

_NC = 2
_NS = 16


def _mesh():
    return plsc.VectorSubcoreMesh(core_axis_name="c", subcore_axis_name="s")


def _chunk_size(n, cap=128):
    for ch in range(cap - cap % 8, 7, -8):
        if n % ch == 0:
            return ch
    raise ValueError(f"no aligned chunk size for {n}")


def _make_deg(N, E):
    n_tiles = _NC * _NS
    ept = E // n_tiles
    assert ept * n_tiles == E and ept % 16 == 0 and N % 16 == 0

    @functools.partial(
        pl.kernel,
        out_type=jax.ShapeDtypeStruct((n_tiles, N), jnp.float32),
        mesh=_mesh(),
        compiler_params=pltpu.CompilerParams(needs_layout_passes=False),
        scratch_types=[
            pltpu.VMEM((ept,), jnp.int32),
            pltpu.VMEM((ept,), jnp.float32),
            pltpu.VMEM((N,), jnp.float32),
        ],
    )
    def deg_kernel(c_hbm, w_hbm, out_hbm, cbuf, wbuf, deg_v):
        tid = lax.axis_index("s") * _NC + lax.axis_index("c")

        def zero_body(i, carry):
            deg_v[pl.ds(i * 16, 16)] = jnp.zeros((16,), jnp.float32)
            return carry

        lax.fori_loop(0, N // 16, zero_body, 0)

        base = tid * ept
        pltpu.sync_copy(c_hbm.at[pl.ds(base, ept)], cbuf)
        pltpu.sync_copy(w_hbm.at[pl.ds(base, ept)], wbuf)

        def body(i, carry):
            idx = cbuf[pl.ds(i * 16, 16)]
            wv = wbuf[pl.ds(i * 16, 16)]
            plsc.addupdate_scatter(deg_v, [idx], wv)
            return carry

        lax.fori_loop(0, ept // 16, body, 0)
        pltpu.sync_copy(deg_v, out_hbm.at[tid])

    return deg_kernel


def _make_msg(N, Dj, cnt, base_a, base_b):
    ept = cnt // _NS
    assert ept * _NS == cnt
    ch = _chunk_size(ept)
    nch = ept // ch
    rpt8 = (N // _NS) // 8 * 8
    extra = N - rpt8 * _NS
    zspan = rpt8 + extra
    zrows = 128
    assert zspan % zrows == 0 and extra % 8 == 0 and Dj % 16 == 0
    nz = zspan // zrows

    @functools.partial(
        pl.kernel,
        out_type=(jax.ShapeDtypeStruct((N, Dj), jnp.float32),
                  jax.ShapeDtypeStruct((N, Dj), jnp.float32)),
        mesh=_mesh(),
        scratch_types=[
            pltpu.VMEM((ch,), jnp.int32),
            pltpu.VMEM((ch,), jnp.int32),
            pltpu.VMEM((ch,), jnp.float32),
            pltpu.VMEM((ch, Dj), jnp.float32),
            pltpu.VMEM((zrows, Dj), jnp.float32),
            pltpu.VMEM_SHARED((N, Dj), jnp.float32),
            pltpu.SemaphoreType.DMA,
        ],
    )
    def msg_kernel(r_hbm, c_hbm, w_hbm, ta_hbm, tb_hbm, oa_hbm, ob_hbm,
                   ridx, cidx, wv, rows, zbuf, acc_sh, sem):
        cid = lax.axis_index("c")
        sid = lax.axis_index("s")

        def zb(i, carry):
            for k in range(Dj // 16):
                zbuf[i, pl.ds(k * 16, 16)] = jnp.zeros((16,), jnp.float32)
            return carry

        lax.fori_loop(0, zrows, zb, 0)
        row0 = pl.multiple_of(sid * rpt8, 8)
        for j in range(nz):
            pltpu.sync_copy(zbuf, acc_sh.at[pl.ds(row0 + j * zrows, zrows)])
        plsc.subcore_barrier()

        def job(t_hbm, ebase):
            base = ebase + sid * ept

            def chunk(g, carry):
                off = base + g * ch
                pltpu.sync_copy(r_hbm.at[pl.ds(off, ch)], ridx)
                pltpu.sync_copy(c_hbm.at[pl.ds(off, ch)], cidx)
                pltpu.sync_copy(w_hbm.at[pl.ds(off, ch)], wv)
                pltpu.async_copy(t_hbm.at[ridx], rows, sem).wait()

                def scale(q, c2):
                    w16 = wv[pl.ds(q * 16, 16)]
                    for j in range(16):
                        s = w16[j]
                        e = q * 16 + j
                        for k in range(Dj // 16):
                            sl = pl.ds(k * 16, 16)
                            rows[e, sl] = rows[e, sl] * s
                    return c2

                lax.fori_loop(0, ch // 16, scale, 0)
                pltpu.sync_copy(rows, acc_sh.at[cidx], add=True)
                return carry

            lax.fori_loop(0, nch, chunk, 0)

        @pl.when(cid == 0)
        def _():
            job(ta_hbm, base_a)

        @pl.when(cid == 1)
        def _():
            job(tb_hbm, base_b)

        plsc.subcore_barrier()

        def copy_out(o_hbm):
            pltpu.sync_copy(acc_sh.at[pl.ds(row0, rpt8)],
                            o_hbm.at[pl.ds(row0, rpt8)])

            @pl.when(sid == _NS - 1)
            def _():
                t0 = N - extra
                pltpu.sync_copy(acc_sh.at[pl.ds(t0, extra)],
                                o_hbm.at[pl.ds(t0, extra)])

        @pl.when(cid == 0)
        def _():
            copy_out(oa_hbm)

        @pl.when(cid == 1)
        def _():
            copy_out(ob_hbm)

    return msg_kernel


def _tc_stage1(deg_parts, x, W1):
    N = x.shape[0]
    Do = W1.shape[1]

    def body(dp_ref, x_ref, w_ref, dinv_ref, xwp_ref):
        deg = jnp.sum(dp_ref[...], axis=0) + 2.0
        dinv = lax.rsqrt(deg)
        dinv_ref[...] = dinv[:, None]
        xw = jnp.dot(x_ref[...], w_ref[...], preferred_element_type=jnp.float32)
        xwp_ref[...] = xw * dinv[:, None]

    return pl.pallas_call(
        body,
        out_shape=(jax.ShapeDtypeStruct((N, 1), jnp.float32),
                   jax.ShapeDtypeStruct((N, Do), jnp.float32)),
    )(deg_parts, x, W1)


def _tc_stage2(acc_lo, acc_hi, xw1p, dinv, b1, W2, W3):
    N, Do = xw1p.shape

    def body(lo_ref, hi_ref, xwp_ref, dv_ref, b_ref, w2_ref, w3_ref,
             o2_ref, o3_ref):
        acc = lo_ref[...] + hi_ref[...]
        dv = dv_ref[...]
        h = jnp.maximum(dv * (acc + 2.0 * xwp_ref[...]) + b_ref[...], 0.0)
        o2_ref[...] = dv * jnp.dot(h, w2_ref[...],
                                   preferred_element_type=jnp.float32)
        o3_ref[...] = dv * jnp.dot(h, w3_ref[...],
                                   preferred_element_type=jnp.float32)

    return pl.pallas_call(
        body,
        out_shape=(jax.ShapeDtypeStruct((N, Do), jnp.float32),
                   jax.ShapeDtypeStruct((N, Do), jnp.float32)),
    )(acc_lo, acc_hi, xw1p, dinv, b1, W2, W3)


def _tc_stage3(acc2, acc3, xw2p, xw3p, dinv, b2, b3):
    N, Do = xw2p.shape

    def body(a2_ref, a3_ref, x2_ref, x3_ref, dv_ref, b2_ref, b3_ref,
             mu_ref, var_ref):
        dv = dv_ref[...]
        mu_ref[...] = dv * (a2_ref[...] + 2.0 * x2_ref[...]) + b2_ref[...]
        var_ref[...] = dv * (a3_ref[...] + 2.0 * x3_ref[...]) + b3_ref[...]

    return pl.pallas_call(
        body,
        out_shape=(jax.ShapeDtypeStruct((N, Do), jnp.float32),
                   jax.ShapeDtypeStruct((N, Do), jnp.float32)),
    )(acc2, acc3, xw2p, xw3p, dinv, b2, b3)


def kernel(x, edge_index, edge_weight, W1, b1, W2, b2, W3, b3):
    N = x.shape[0]
    E = edge_weight.shape[0]
    Do = W1.shape[1]
    r = edge_index[0]
    c = edge_index[1]

    deg_parts = _make_deg(N, E)(c, edge_weight)
    dinv, xw1p = _tc_stage1(deg_parts, x, W1)

    acc_a, acc_b = _make_msg(N, Do, E // 2, 0, E // 2)(
        r, c, edge_weight, xw1p, xw1p)

    xw2p, xw3p = _tc_stage2(acc_a, acc_b, xw1p, dinv,
                            b1.reshape(1, -1), W2, W3)

    acc2, acc3 = _make_msg(N, Do, E, 0, 0)(r, c, edge_weight, xw2p, xw3p)

    mu, var = _tc_stage3(acc2, acc3, xw2p, xw3p, dinv,
                         b2.reshape(1, -1), b3.reshape(1, -1))
    return (mu, var)


# speedup vs baseline: 23.9744x; 2.5391x over previous
import functools

import jax
import jax.numpy as jnp
from jax import lax
from jax.experimental import pallas as pl
from jax.experimental.pallas import tpu as pltpu
from jax.experimental.pallas import tpu_sc as plsc

_NC = 2
_NS = 16


def _mesh():
    return plsc.VectorSubcoreMesh(core_axis_name="c", subcore_axis_name="s")


def _chunk_size(n, cap=128):
    for ch in range(cap - cap % 8, 7, -8):
        if n % ch == 0:
            return ch
    raise ValueError(f"no aligned chunk size for {n}")


def _make_deg(N, E):
    n_tiles = _NC * _NS
    ept = E // n_tiles
    assert ept * n_tiles == E and ept % 16 == 0 and N % 16 == 0

    @functools.partial(
        pl.kernel,
        out_type=jax.ShapeDtypeStruct((n_tiles, N), jnp.float32),
        mesh=_mesh(),
        compiler_params=pltpu.CompilerParams(needs_layout_passes=False),
        scratch_types=[
            pltpu.VMEM((ept,), jnp.int32),
            pltpu.VMEM((ept,), jnp.float32),
            pltpu.VMEM((N,), jnp.float32),
        ],
    )
    def deg_kernel(c_hbm, w_hbm, out_hbm, cbuf, wbuf, deg_v):
        tid = lax.axis_index("s") * _NC + lax.axis_index("c")

        def zero_body(i, carry):
            deg_v[pl.ds(i * 16, 16)] = jnp.zeros((16,), jnp.float32)
            return carry

        lax.fori_loop(0, N // 16, zero_body, 0)

        base = tid * ept
        pltpu.sync_copy(c_hbm.at[pl.ds(base, ept)], cbuf)
        pltpu.sync_copy(w_hbm.at[pl.ds(base, ept)], wbuf)

        def body(i, carry):
            idx = cbuf[pl.ds(i * 16, 16)]
            wv = wbuf[pl.ds(i * 16, 16)]
            plsc.addupdate_scatter(deg_v, [idx], wv)
            return carry

        lax.fori_loop(0, ept // 16, body, 0)
        pltpu.sync_copy(deg_v, out_hbm.at[tid])

    return deg_kernel


def _make_msg(N, Dj, cnt, base_a, base_b):
    ept = cnt // _NS
    assert ept * _NS == cnt
    ch = _chunk_size(ept)
    assert ch % 16 == 0
    nch = ept // ch
    assert nch >= 4
    rpt8 = (N // _NS) // 8 * 8
    extra = N - rpt8 * _NS
    zspan = rpt8 + extra
    zrows = 64
    assert zspan % zrows == 0 and extra % 8 == 0 and Dj % 16 == 0
    nz = zspan // zrows

    @functools.partial(
        pl.kernel,
        out_type=(jax.ShapeDtypeStruct((N, Dj), jnp.float32),
                  jax.ShapeDtypeStruct((N, Dj), jnp.float32)),
        mesh=_mesh(),
        scratch_types=[
            pltpu.VMEM((ch,), jnp.int32),
            pltpu.VMEM((ch,), jnp.int32),
            pltpu.VMEM((ch,), jnp.int32),
            pltpu.VMEM((ch,), jnp.int32),
            pltpu.VMEM((ch,), jnp.float32),
            pltpu.VMEM((ch,), jnp.float32),
            pltpu.VMEM((ch, Dj), jnp.float32),
            pltpu.VMEM((ch, Dj), jnp.float32),
            pltpu.VMEM((zrows, Dj), jnp.float32),
            pltpu.VMEM_SHARED((N, Dj), jnp.float32),
            pltpu.SemaphoreType.DMA,
            pltpu.SemaphoreType.DMA,
            pltpu.SemaphoreType.DMA,
        ],
    )
    def msg_kernel(r_hbm, c_hbm, w_hbm, ta_hbm, tb_hbm, oa_hbm, ob_hbm,
                   ridx0, ridx1, cidx0, cidx1, wv0, wv1, rows0, rows1,
                   zbuf, acc_sh, isem, gsem, ssem):
        cid = lax.axis_index("c")
        sid = lax.axis_index("s")
        ridx_bufs = (ridx0, ridx1)
        cidx_bufs = (cidx0, cidx1)
        wv_bufs = (wv0, wv1)
        rows_bufs = (rows0, rows1)

        def zb(i, carry):
            for k in range(Dj // 16):
                zbuf[i, pl.ds(k * 16, 16)] = jnp.zeros((16,), jnp.float32)
            return carry

        lax.fori_loop(0, zrows, zb, 0)
        row0 = pl.multiple_of(sid * rpt8, 8)
        for j in range(nz):
            pltpu.sync_copy(zbuf, acc_sh.at[pl.ds(row0 + j * zrows, zrows)])
        plsc.subcore_barrier()

        def job(t_hbm, ebase):
            base = ebase + sid * ept

            def start_idx(g, b):
                off = base + g * ch
                pltpu.async_copy(r_hbm.at[pl.ds(off, ch)], ridx_bufs[b], isem)
                pltpu.async_copy(c_hbm.at[pl.ds(off, ch)], cidx_bufs[b], isem)
                pltpu.async_copy(w_hbm.at[pl.ds(off, ch)], wv_bufs[b], isem)

            def wait_idx(g, b):
                off = base + g * ch
                pltpu.make_async_copy(
                    r_hbm.at[pl.ds(off, ch)], ridx_bufs[b], isem).wait()
                pltpu.make_async_copy(
                    c_hbm.at[pl.ds(off, ch)], cidx_bufs[b], isem).wait()
                pltpu.make_async_copy(
                    w_hbm.at[pl.ds(off, ch)], wv_bufs[b], isem).wait()

            def start_gather(b):
                pltpu.async_copy(t_hbm.at[ridx_bufs[b]], rows_bufs[b], gsem)

            def wait_gather(b):
                pltpu.make_async_copy(
                    t_hbm.at[ridx_bufs[b]], rows_bufs[b], gsem).wait()

            def scale(b):
                rows = rows_bufs[b]
                wv = wv_bufs[b]

                def sbody(q, c2):
                    w16 = wv[pl.ds(q * 16, 16)]
                    for j in range(16):
                        s = w16[j]
                        e = q * 16 + j
                        for k in range(Dj // 16):
                            sl = pl.ds(k * 16, 16)
                            rows[e, sl] = rows[e, sl] * s
                    return c2

                lax.fori_loop(0, ch // 16, sbody, 0)

            def issue_scatter(b):
                for q in range(ch // 16):
                    cvec = cidx_bufs[b][pl.ds(q * 16, 16)]
                    pltpu.async_copy(rows_bufs[b].at[pl.ds(q * 16, 16)],
                                     acc_sh.at[cvec], ssem, add=True)

            def drain_scatter(b):
                for q in range(ch // 16):
                    cvec = cidx_bufs[b][pl.ds(q * 16, 16)]
                    pltpu.make_async_copy(
                        rows_bufs[b].at[pl.ds(q * 16, 16)],
                        acc_sh.at[cvec], ssem).wait()

            def iteration(g, b, has_next, has_next2, has_prev):
                nb = 1 - b
                if has_next:
                    wait_idx(g + 1, nb)
                    if has_prev:
                        drain_scatter(nb)
                    start_gather(nb)
                elif has_prev:
                    drain_scatter(nb)
                wait_gather(b)
                scale(b)
                issue_scatter(b)
                if has_next2:
                    start_idx(g + 2, b)

            start_idx(0, 0)
            wait_idx(0, 0)
            start_gather(0)
            start_idx(1, 1)
            iteration(0, 0, True, nch > 2, False)

            npairs = (nch - 3) // 2

            def pair(p, carry):
                g = 1 + 2 * p
                iteration(g, 1, True, True, True)
                iteration(g + 1, 0, True, True, True)
                return carry

            lax.fori_loop(0, npairs, pair, 0)

            for g in range(1 + 2 * npairs, nch):
                iteration(g, g % 2, g + 1 < nch, g + 2 < nch, True)
            drain_scatter((nch - 1) % 2)

        @pl.when(cid == 0)
        def _():
            job(ta_hbm, base_a)

        @pl.when(cid == 1)
        def _():
            job(tb_hbm, base_b)

        plsc.subcore_barrier()

        def copy_out(o_hbm):
            pltpu.sync_copy(acc_sh.at[pl.ds(row0, rpt8)],
                            o_hbm.at[pl.ds(row0, rpt8)])

            @pl.when(sid == _NS - 1)
            def _():
                t0 = N - extra
                pltpu.sync_copy(acc_sh.at[pl.ds(t0, extra)],
                                o_hbm.at[pl.ds(t0, extra)])

        @pl.when(cid == 0)
        def _():
            copy_out(oa_hbm)

        @pl.when(cid == 1)
        def _():
            copy_out(ob_hbm)

    return msg_kernel


def _tc_stage1(deg_parts, x, W1):
    N = x.shape[0]
    Do = W1.shape[1]

    def body(dp_ref, x_ref, w_ref, dinv_ref, xwp_ref):
        deg = jnp.sum(dp_ref[...], axis=0) + 2.0
        dinv = lax.rsqrt(deg)
        dinv_ref[...] = dinv[:, None]
        xw = jnp.dot(x_ref[...], w_ref[...], preferred_element_type=jnp.float32)
        xwp_ref[...] = xw * dinv[:, None]

    return pl.pallas_call(
        body,
        out_shape=(jax.ShapeDtypeStruct((N, 1), jnp.float32),
                   jax.ShapeDtypeStruct((N, Do), jnp.float32)),
    )(deg_parts, x, W1)


def _tc_stage2(acc_lo, acc_hi, xw1p, dinv, b1, W2, W3):
    N, Do = xw1p.shape

    def body(lo_ref, hi_ref, xwp_ref, dv_ref, b_ref, w2_ref, w3_ref,
             o2_ref, o3_ref):
        acc = lo_ref[...] + hi_ref[...]
        dv = dv_ref[...]
        h = jnp.maximum(dv * (acc + 2.0 * xwp_ref[...]) + b_ref[...], 0.0)
        o2_ref[...] = dv * jnp.dot(h, w2_ref[...],
                                   preferred_element_type=jnp.float32)
        o3_ref[...] = dv * jnp.dot(h, w3_ref[...],
                                   preferred_element_type=jnp.float32)

    return pl.pallas_call(
        body,
        out_shape=(jax.ShapeDtypeStruct((N, Do), jnp.float32),
                   jax.ShapeDtypeStruct((N, Do), jnp.float32)),
    )(acc_lo, acc_hi, xw1p, dinv, b1, W2, W3)


def _tc_stage3(acc2, acc3, xw2p, xw3p, dinv, b2, b3):
    N, Do = xw2p.shape

    def body(a2_ref, a3_ref, x2_ref, x3_ref, dv_ref, b2_ref, b3_ref,
             mu_ref, var_ref):
        dv = dv_ref[...]
        mu_ref[...] = dv * (a2_ref[...] + 2.0 * x2_ref[...]) + b2_ref[...]
        var_ref[...] = dv * (a3_ref[...] + 2.0 * x3_ref[...]) + b3_ref[...]

    return pl.pallas_call(
        body,
        out_shape=(jax.ShapeDtypeStruct((N, Do), jnp.float32),
                   jax.ShapeDtypeStruct((N, Do), jnp.float32)),
    )(acc2, acc3, xw2p, xw3p, dinv, b2, b3)


def kernel(x, edge_index, edge_weight, W1, b1, W2, b2, W3, b3):
    N = x.shape[0]
    E = edge_weight.shape[0]
    Do = W1.shape[1]
    r = edge_index[0]
    c = edge_index[1]

    deg_parts = _make_deg(N, E)(c, edge_weight)
    dinv, xw1p = _tc_stage1(deg_parts, x, W1)

    acc_a, acc_b = _make_msg(N, Do, E // 2, 0, E // 2)(
        r, c, edge_weight, xw1p, xw1p)

    xw2p, xw3p = _tc_stage2(acc_a, acc_b, xw1p, dinv,
                            b1.reshape(1, -1), W2, W3)

    acc2, acc3 = _make_msg(N, Do, E, 0, 0)(r, c, edge_weight, xw2p, xw3p)

    mu, var = _tc_stage3(acc2, acc3, xw2p, xw3p, dinv,
                         b2.reshape(1, -1), b3.reshape(1, -1))
    return (mu, var)


# speedup vs baseline: 24.8693x; 1.0373x over previous
import functools

import jax
import jax.numpy as jnp
from jax import lax
from jax.experimental import pallas as pl
from jax.experimental.pallas import tpu as pltpu
from jax.experimental.pallas import tpu_sc as plsc

_NC = 2
_NS = 16


def _mesh():
    return plsc.VectorSubcoreMesh(core_axis_name="c", subcore_axis_name="s")


def _chunk_size(n, cap=128):
    for ch in range(cap - cap % 8, 7, -8):
        if n % ch == 0:
            return ch
    raise ValueError(f"no aligned chunk size for {n}")


def _make_deg(N, E):
    n_tiles = _NC * _NS
    ept = E // n_tiles
    assert ept * n_tiles == E and ept % 16 == 0 and N % 16 == 0

    @functools.partial(
        pl.kernel,
        out_type=jax.ShapeDtypeStruct((n_tiles, N), jnp.float32),
        mesh=_mesh(),
        compiler_params=pltpu.CompilerParams(needs_layout_passes=False),
        scratch_types=[
            pltpu.VMEM((ept,), jnp.int32),
            pltpu.VMEM((ept,), jnp.float32),
            pltpu.VMEM((N,), jnp.float32),
        ],
    )
    def deg_kernel(c_hbm, w_hbm, out_hbm, cbuf, wbuf, deg_v):
        tid = lax.axis_index("s") * _NC + lax.axis_index("c")

        def zero_body(i, carry):
            deg_v[pl.ds(i * 16, 16)] = jnp.zeros((16,), jnp.float32)
            return carry

        lax.fori_loop(0, N // 16, zero_body, 0)

        base = tid * ept
        pltpu.sync_copy(c_hbm.at[pl.ds(base, ept)], cbuf)
        pltpu.sync_copy(w_hbm.at[pl.ds(base, ept)], wbuf)

        def body(i, carry):
            idx = cbuf[pl.ds(i * 16, 16)]
            wv = wbuf[pl.ds(i * 16, 16)]
            plsc.addupdate_scatter(deg_v, [idx], wv)
            return carry

        lax.fori_loop(0, ept // 16, body, 0)
        pltpu.sync_copy(deg_v, out_hbm.at[tid])

    return deg_kernel


def _make_msg(N, Dj, cnt, base_a, base_b):
    ept = cnt // _NS
    assert ept * _NS == cnt
    ch = _chunk_size(ept)
    assert ch % 16 == 0
    nch = ept // ch
    assert nch >= 4
    rpt8 = (N // _NS) // 8 * 8
    extra = N - rpt8 * _NS
    zspan = rpt8 + extra
    zrows = 64
    assert zspan % zrows == 0 and extra % 8 == 0 and Dj % 16 == 0
    nz = zspan // zrows

    @functools.partial(
        pl.kernel,
        out_type=(jax.ShapeDtypeStruct((N, Dj), jnp.float32),
                  jax.ShapeDtypeStruct((N, Dj), jnp.float32)),
        mesh=_mesh(),
        scratch_types=[
            pltpu.VMEM((ch,), jnp.int32),
            pltpu.VMEM((ch,), jnp.int32),
            pltpu.VMEM((ch,), jnp.int32),
            pltpu.VMEM((ch,), jnp.int32),
            pltpu.VMEM((ch,), jnp.int32),
            pltpu.VMEM((ch,), jnp.int32),
            pltpu.VMEM((ch,), jnp.float32),
            pltpu.VMEM((ch,), jnp.float32),
            pltpu.VMEM((ch,), jnp.float32),
            pltpu.VMEM((ch, Dj), jnp.float32),
            pltpu.VMEM((ch, Dj), jnp.float32),
            pltpu.VMEM((ch, Dj), jnp.float32),
            pltpu.VMEM((zrows, Dj), jnp.float32),
            pltpu.VMEM_SHARED((N, Dj), jnp.float32),
            pltpu.SemaphoreType.DMA,
            pltpu.SemaphoreType.DMA,
            pltpu.SemaphoreType.DMA,
        ],
    )
    def msg_kernel(r_hbm, c_hbm, w_hbm, ta_hbm, tb_hbm, oa_hbm, ob_hbm,
                   ridx0, ridx1, ridx2, cidx0, cidx1, cidx2, wv0, wv1, wv2,
                   rows0, rows1, rows2, zbuf, acc_sh, isem, gsem, ssem):
        cid = lax.axis_index("c")
        sid = lax.axis_index("s")
        ridx_bufs = (ridx0, ridx1, ridx2)
        cidx_bufs = (cidx0, cidx1, cidx2)
        wv_bufs = (wv0, wv1, wv2)
        rows_bufs = (rows0, rows1, rows2)

        def zb(i, carry):
            for k in range(Dj // 16):
                zbuf[i, pl.ds(k * 16, 16)] = jnp.zeros((16,), jnp.float32)
            return carry

        lax.fori_loop(0, zrows, zb, 0)
        row0 = pl.multiple_of(sid * rpt8, 8)
        for j in range(nz):
            pltpu.sync_copy(zbuf, acc_sh.at[pl.ds(row0 + j * zrows, zrows)])
        plsc.subcore_barrier()

        def job(t_hbm, ebase):
            base = ebase + sid * ept

            def start_idx(g, b):
                off = base + g * ch
                pltpu.async_copy(r_hbm.at[pl.ds(off, ch)], ridx_bufs[b], isem)
                pltpu.async_copy(c_hbm.at[pl.ds(off, ch)], cidx_bufs[b], isem)
                pltpu.async_copy(w_hbm.at[pl.ds(off, ch)], wv_bufs[b], isem)

            def wait_idx(g, b):
                off = base + g * ch
                pltpu.make_async_copy(
                    r_hbm.at[pl.ds(off, ch)], ridx_bufs[b], isem).wait()
                pltpu.make_async_copy(
                    c_hbm.at[pl.ds(off, ch)], cidx_bufs[b], isem).wait()
                pltpu.make_async_copy(
                    w_hbm.at[pl.ds(off, ch)], wv_bufs[b], isem).wait()

            def start_gather(b):
                pltpu.async_copy(t_hbm.at[ridx_bufs[b]], rows_bufs[b], gsem)

            def wait_gather(b):
                pltpu.make_async_copy(
                    t_hbm.at[ridx_bufs[b]], rows_bufs[b], gsem).wait()

            def scale(b):
                rows = rows_bufs[b]
                wv = wv_bufs[b]

                def sbody(q, c2):
                    w16 = wv[pl.ds(q * 16, 16)]
                    for j in range(16):
                        s = w16[j]
                        e = q * 16 + j
                        for k in range(Dj // 16):
                            sl = pl.ds(k * 16, 16)
                            rows[e, sl] = rows[e, sl] * s
                    return c2

                lax.fori_loop(0, ch // 16, sbody, 0)

            def issue_scatter(b):
                for q in range(ch // 16):
                    cvec = cidx_bufs[b][pl.ds(q * 16, 16)]
                    pltpu.async_copy(rows_bufs[b].at[pl.ds(q * 16, 16)],
                                     acc_sh.at[cvec], ssem, add=True)

            def drain_scatter(b):
                for q in range(ch // 16):
                    cvec = cidx_bufs[b][pl.ds(q * 16, 16)]
                    pltpu.make_async_copy(
                        rows_bufs[b].at[pl.ds(q * 16, 16)],
                        acc_sh.at[cvec], ssem).wait()

            def iteration(g, b, has_prev, has_n2, has_n3):
                b2 = (b + 2) % 3
                if has_n2:
                    wait_idx(g + 2, b2)
                    if has_prev:
                        drain_scatter(b2)
                    start_gather(b2)
                elif has_prev:
                    drain_scatter(b2)
                wait_gather(b)
                scale(b)
                issue_scatter(b)
                if has_n3:
                    start_idx(g + 3, b)

            start_idx(0, 0)
            wait_idx(0, 0)
            start_gather(0)
            start_idx(1, 1)
            wait_idx(1, 1)
            start_gather(1)
            start_idx(2, 2)
            iteration(0, 0, False, nch > 2, nch > 3)

            ntriples = (nch - 4) // 3

            def triple(p, carry):
                g = 1 + 3 * p
                iteration(g, 1, True, True, True)
                iteration(g + 1, 2, True, True, True)
                iteration(g + 2, 0, True, True, True)
                return carry

            lax.fori_loop(0, ntriples, triple, 0)

            for g in range(1 + 3 * ntriples, nch):
                iteration(g, g % 3, True, g + 2 < nch, g + 3 < nch)
            drain_scatter((nch - 1) % 3)

        @pl.when(cid == 0)
        def _():
            job(ta_hbm, base_a)

        @pl.when(cid == 1)
        def _():
            job(tb_hbm, base_b)

        plsc.subcore_barrier()

        def copy_out(o_hbm):
            pltpu.sync_copy(acc_sh.at[pl.ds(row0, rpt8)],
                            o_hbm.at[pl.ds(row0, rpt8)])

            @pl.when(sid == _NS - 1)
            def _():
                t0 = N - extra
                pltpu.sync_copy(acc_sh.at[pl.ds(t0, extra)],
                                o_hbm.at[pl.ds(t0, extra)])

        @pl.when(cid == 0)
        def _():
            copy_out(oa_hbm)

        @pl.when(cid == 1)
        def _():
            copy_out(ob_hbm)

    return msg_kernel


def _tc_stage1(deg_parts, x, W1):
    N = x.shape[0]
    Do = W1.shape[1]

    def body(dp_ref, x_ref, w_ref, dinv_ref, xwp_ref):
        deg = jnp.sum(dp_ref[...], axis=0) + 2.0
        dinv = lax.rsqrt(deg)
        dinv_ref[...] = dinv[:, None]
        xw = jnp.dot(x_ref[...], w_ref[...], preferred_element_type=jnp.float32)
        xwp_ref[...] = xw * dinv[:, None]

    return pl.pallas_call(
        body,
        out_shape=(jax.ShapeDtypeStruct((N, 1), jnp.float32),
                   jax.ShapeDtypeStruct((N, Do), jnp.float32)),
    )(deg_parts, x, W1)


def _tc_stage2(acc_lo, acc_hi, xw1p, dinv, b1, W2, W3):
    N, Do = xw1p.shape

    def body(lo_ref, hi_ref, xwp_ref, dv_ref, b_ref, w2_ref, w3_ref,
             o2_ref, o3_ref):
        acc = lo_ref[...] + hi_ref[...]
        dv = dv_ref[...]
        h = jnp.maximum(dv * (acc + 2.0 * xwp_ref[...]) + b_ref[...], 0.0)
        o2_ref[...] = dv * jnp.dot(h, w2_ref[...],
                                   preferred_element_type=jnp.float32)
        o3_ref[...] = dv * jnp.dot(h, w3_ref[...],
                                   preferred_element_type=jnp.float32)

    return pl.pallas_call(
        body,
        out_shape=(jax.ShapeDtypeStruct((N, Do), jnp.float32),
                   jax.ShapeDtypeStruct((N, Do), jnp.float32)),
    )(acc_lo, acc_hi, xw1p, dinv, b1, W2, W3)


def _tc_stage3(acc2, acc3, xw2p, xw3p, dinv, b2, b3):
    N, Do = xw2p.shape

    def body(a2_ref, a3_ref, x2_ref, x3_ref, dv_ref, b2_ref, b3_ref,
             mu_ref, var_ref):
        dv = dv_ref[...]
        mu_ref[...] = dv * (a2_ref[...] + 2.0 * x2_ref[...]) + b2_ref[...]
        var_ref[...] = dv * (a3_ref[...] + 2.0 * x3_ref[...]) + b3_ref[...]

    return pl.pallas_call(
        body,
        out_shape=(jax.ShapeDtypeStruct((N, Do), jnp.float32),
                   jax.ShapeDtypeStruct((N, Do), jnp.float32)),
    )(acc2, acc3, xw2p, xw3p, dinv, b2, b3)


def kernel(x, edge_index, edge_weight, W1, b1, W2, b2, W3, b3):
    N = x.shape[0]
    E = edge_weight.shape[0]
    Do = W1.shape[1]
    r = edge_index[0]
    c = edge_index[1]

    deg_parts = _make_deg(N, E)(c, edge_weight)
    dinv, xw1p = _tc_stage1(deg_parts, x, W1)

    acc_a, acc_b = _make_msg(N, Do, E // 2, 0, E // 2)(
        r, c, edge_weight, xw1p, xw1p)

    xw2p, xw3p = _tc_stage2(acc_a, acc_b, xw1p, dinv,
                            b1.reshape(1, -1), W2, W3)

    acc2, acc3 = _make_msg(N, Do, E, 0, 0)(r, c, edge_weight, xw2p, xw3p)

    mu, var = _tc_stage3(acc2, acc3, xw2p, xw3p, dinv,
                         b2.reshape(1, -1), b3.reshape(1, -1))
    return (mu, var)


# speedup vs baseline: 24.9869x; 1.0047x over previous
import functools

import jax
import jax.numpy as jnp
from jax import lax
from jax.experimental import pallas as pl
from jax.experimental.pallas import tpu as pltpu
from jax.experimental.pallas import tpu_sc as plsc

_NC = 2
_NS = 16


def _mesh():
    return plsc.VectorSubcoreMesh(core_axis_name="c", subcore_axis_name="s")


def _chunk_size(n, cap=128):
    for ch in range(cap - cap % 8, 7, -8):
        if n % ch == 0:
            return ch
    raise ValueError(f"no aligned chunk size for {n}")


def _make_deg(N, E):
    n_tiles = _NC * _NS
    ept = E // n_tiles
    assert ept * n_tiles == E and ept % 16 == 0 and N % 16 == 0

    @functools.partial(
        pl.kernel,
        out_type=jax.ShapeDtypeStruct((n_tiles, N), jnp.float32),
        mesh=_mesh(),
        compiler_params=pltpu.CompilerParams(needs_layout_passes=False),
        scratch_types=[
            pltpu.VMEM((ept,), jnp.int32),
            pltpu.VMEM((ept,), jnp.float32),
            pltpu.VMEM((N,), jnp.float32),
            pltpu.SemaphoreType.DMA,
        ],
    )
    def deg_kernel(c_hbm, w_hbm, out_hbm, cbuf, wbuf, deg_v, sem):
        tid = lax.axis_index("s") * _NC + lax.axis_index("c")
        base = tid * ept
        pltpu.async_copy(c_hbm.at[pl.ds(base, ept)], cbuf, sem)
        pltpu.async_copy(w_hbm.at[pl.ds(base, ept)], wbuf, sem)

        def zero_body(i, carry):
            deg_v[pl.ds(i * 16, 16)] = jnp.zeros((16,), jnp.float32)
            return carry

        lax.fori_loop(0, N // 16, zero_body, 0)
        pltpu.make_async_copy(c_hbm.at[pl.ds(base, ept)], cbuf, sem).wait()
        pltpu.make_async_copy(w_hbm.at[pl.ds(base, ept)], wbuf, sem).wait()

        def body(i, carry):
            idx = cbuf[pl.ds(i * 16, 16)]
            wv = wbuf[pl.ds(i * 16, 16)]
            plsc.addupdate_scatter(deg_v, [idx], wv)
            return carry

        lax.fori_loop(0, ept // 16, body, 0)
        pltpu.sync_copy(deg_v, out_hbm.at[tid])

    return deg_kernel


def _make_msg(N, Dj, cnt, base_a, base_b):
    ept = cnt // _NS
    assert ept * _NS == cnt
    ch = _chunk_size(ept)
    assert ch % 16 == 0
    nch = ept // ch
    assert nch >= 4
    rpt8 = (N // _NS) // 8 * 8
    extra = N - rpt8 * _NS
    zspan = rpt8 + extra
    zrows = 64
    assert zspan % zrows == 0 and extra % 8 == 0 and Dj % 16 == 0
    nz = zspan // zrows

    @functools.partial(
        pl.kernel,
        out_type=(jax.ShapeDtypeStruct((N, Dj), jnp.float32),
                  jax.ShapeDtypeStruct((N, Dj), jnp.float32)),
        mesh=_mesh(),
        scratch_types=[
            pltpu.VMEM((ch,), jnp.int32),
            pltpu.VMEM((ch,), jnp.int32),
            pltpu.VMEM((ch,), jnp.int32),
            pltpu.VMEM((ch,), jnp.int32),
            pltpu.VMEM((ch,), jnp.int32),
            pltpu.VMEM((ch,), jnp.int32),
            pltpu.VMEM((ch,), jnp.float32),
            pltpu.VMEM((ch,), jnp.float32),
            pltpu.VMEM((ch,), jnp.float32),
            pltpu.VMEM((ch, Dj), jnp.float32),
            pltpu.VMEM((ch, Dj), jnp.float32),
            pltpu.VMEM((ch, Dj), jnp.float32),
            pltpu.VMEM((zrows, Dj), jnp.float32),
            pltpu.VMEM_SHARED((N, Dj), jnp.float32),
            pltpu.SemaphoreType.DMA,
            pltpu.SemaphoreType.DMA,
            pltpu.SemaphoreType.DMA,
        ],
    )
    def msg_kernel(r_hbm, c_hbm, w_hbm, ta_hbm, tb_hbm, oa_hbm, ob_hbm,
                   ridx0, ridx1, ridx2, cidx0, cidx1, cidx2, wv0, wv1, wv2,
                   rows0, rows1, rows2, zbuf, acc_sh, isem, gsem, ssem):
        cid = lax.axis_index("c")
        sid = lax.axis_index("s")
        ridx_bufs = (ridx0, ridx1, ridx2)
        cidx_bufs = (cidx0, cidx1, cidx2)
        wv_bufs = (wv0, wv1, wv2)
        rows_bufs = (rows0, rows1, rows2)

        def start_idx_at(base, g, b):
            off = base + g * ch
            pltpu.async_copy(r_hbm.at[pl.ds(off, ch)], ridx_bufs[b], isem)
            pltpu.async_copy(c_hbm.at[pl.ds(off, ch)], cidx_bufs[b], isem)
            pltpu.async_copy(w_hbm.at[pl.ds(off, ch)], wv_bufs[b], isem)

        @pl.when(cid == 0)
        def _():
            for g in range(3):
                start_idx_at(base_a + sid * ept, g, g)

        @pl.when(cid == 1)
        def _():
            for g in range(3):
                start_idx_at(base_b + sid * ept, g, g)

        def zb(i, carry):
            for k in range(Dj // 16):
                zbuf[i, pl.ds(k * 16, 16)] = jnp.zeros((16,), jnp.float32)
            return carry

        lax.fori_loop(0, zrows, zb, 0)
        row0 = pl.multiple_of(sid * rpt8, 8)
        for j in range(nz):
            pltpu.async_copy(zbuf, acc_sh.at[pl.ds(row0 + j * zrows, zrows)],
                             ssem)
        for j in range(nz):
            pltpu.make_async_copy(
                zbuf, acc_sh.at[pl.ds(row0 + j * zrows, zrows)], ssem).wait()

        def job(t_hbm, ebase):
            base = ebase + sid * ept

            def start_idx(g, b):
                start_idx_at(base, g, b)

            def wait_idx(g, b):
                off = base + g * ch
                pltpu.make_async_copy(
                    r_hbm.at[pl.ds(off, ch)], ridx_bufs[b], isem).wait()
                pltpu.make_async_copy(
                    c_hbm.at[pl.ds(off, ch)], cidx_bufs[b], isem).wait()
                pltpu.make_async_copy(
                    w_hbm.at[pl.ds(off, ch)], wv_bufs[b], isem).wait()

            def start_gather(b):
                pltpu.async_copy(t_hbm.at[ridx_bufs[b]], rows_bufs[b], gsem)

            def wait_gather(b):
                pltpu.make_async_copy(
                    t_hbm.at[ridx_bufs[b]], rows_bufs[b], gsem).wait()

            def scale(b):
                rows = rows_bufs[b]
                wv = wv_bufs[b]

                def sbody(q, c2):
                    w16 = wv[pl.ds(q * 16, 16)]
                    for j in range(16):
                        s = w16[j]
                        e = q * 16 + j
                        for k in range(Dj // 16):
                            sl = pl.ds(k * 16, 16)
                            rows[e, sl] = rows[e, sl] * s
                    return c2

                lax.fori_loop(0, ch // 16, sbody, 0)

            def issue_scatter(b):
                for q in range(ch // 16):
                    cvec = cidx_bufs[b][pl.ds(q * 16, 16)]
                    pltpu.async_copy(rows_bufs[b].at[pl.ds(q * 16, 16)],
                                     acc_sh.at[cvec], ssem, add=True)

            def drain_scatter(b):
                for q in range(ch // 16):
                    cvec = cidx_bufs[b][pl.ds(q * 16, 16)]
                    pltpu.make_async_copy(
                        rows_bufs[b].at[pl.ds(q * 16, 16)],
                        acc_sh.at[cvec], ssem).wait()

            def iteration(g, b, has_prev, has_n2, has_n3):
                b2 = (b + 2) % 3
                if has_n2:
                    wait_idx(g + 2, b2)
                    if has_prev:
                        drain_scatter(b2)
                    start_gather(b2)
                elif has_prev:
                    drain_scatter(b2)
                wait_gather(b)
                scale(b)
                issue_scatter(b)
                if has_n3:
                    start_idx(g + 3, b)

            wait_idx(0, 0)
            start_gather(0)
            wait_idx(1, 1)
            start_gather(1)
            plsc.subcore_barrier()
            iteration(0, 0, False, nch > 2, nch > 3)

            ntriples = (nch - 4) // 3

            def triple(p, carry):
                g = 1 + 3 * p
                iteration(g, 1, True, True, True)
                iteration(g + 1, 2, True, True, True)
                iteration(g + 2, 0, True, True, True)
                return carry

            lax.fori_loop(0, ntriples, triple, 0)

            for g in range(1 + 3 * ntriples, nch):
                iteration(g, g % 3, True, g + 2 < nch, g + 3 < nch)
            drain_scatter((nch - 1) % 3)

        @pl.when(cid == 0)
        def _():
            job(ta_hbm, base_a)

        @pl.when(cid == 1)
        def _():
            job(tb_hbm, base_b)

        plsc.subcore_barrier()

        def copy_out(o_hbm):
            pltpu.sync_copy(acc_sh.at[pl.ds(row0, rpt8)],
                            o_hbm.at[pl.ds(row0, rpt8)])

            @pl.when(sid == _NS - 1)
            def _():
                t0 = N - extra
                pltpu.sync_copy(acc_sh.at[pl.ds(t0, extra)],
                                o_hbm.at[pl.ds(t0, extra)])

        @pl.when(cid == 0)
        def _():
            copy_out(oa_hbm)

        @pl.when(cid == 1)
        def _():
            copy_out(ob_hbm)

    return msg_kernel


def _tc_stage1(deg_parts, x, W1):
    N = x.shape[0]
    Do = W1.shape[1]

    def body(dp_ref, x_ref, w_ref, dinv_ref, xwp_ref):
        deg = jnp.sum(dp_ref[...], axis=0) + 2.0
        dinv = lax.rsqrt(deg)
        dinv_ref[...] = dinv[:, None]
        xw = jnp.dot(x_ref[...], w_ref[...], preferred_element_type=jnp.float32)
        xwp_ref[...] = xw * dinv[:, None]

    return pl.pallas_call(
        body,
        out_shape=(jax.ShapeDtypeStruct((N, 1), jnp.float32),
                   jax.ShapeDtypeStruct((N, Do), jnp.float32)),
    )(deg_parts, x, W1)


def _tc_stage2(acc_lo, acc_hi, xw1p, dinv, b1, W2, W3):
    N, Do = xw1p.shape

    def body(lo_ref, hi_ref, xwp_ref, dv_ref, b_ref, w2_ref, w3_ref,
             o2_ref, o3_ref):
        acc = lo_ref[...] + hi_ref[...]
        dv = dv_ref[...]
        h = jnp.maximum(dv * (acc + 2.0 * xwp_ref[...]) + b_ref[...], 0.0)
        o2_ref[...] = dv * jnp.dot(h, w2_ref[...],
                                   preferred_element_type=jnp.float32)
        o3_ref[...] = dv * jnp.dot(h, w3_ref[...],
                                   preferred_element_type=jnp.float32)

    return pl.pallas_call(
        body,
        out_shape=(jax.ShapeDtypeStruct((N, Do), jnp.float32),
                   jax.ShapeDtypeStruct((N, Do), jnp.float32)),
    )(acc_lo, acc_hi, xw1p, dinv, b1, W2, W3)


def _tc_stage3(acc2, acc3, xw2p, xw3p, dinv, b2, b3):
    N, Do = xw2p.shape

    def body(a2_ref, a3_ref, x2_ref, x3_ref, dv_ref, b2_ref, b3_ref,
             mu_ref, var_ref):
        dv = dv_ref[...]
        mu_ref[...] = dv * (a2_ref[...] + 2.0 * x2_ref[...]) + b2_ref[...]
        var_ref[...] = dv * (a3_ref[...] + 2.0 * x3_ref[...]) + b3_ref[...]

    return pl.pallas_call(
        body,
        out_shape=(jax.ShapeDtypeStruct((N, Do), jnp.float32),
                   jax.ShapeDtypeStruct((N, Do), jnp.float32)),
    )(acc2, acc3, xw2p, xw3p, dinv, b2, b3)


def kernel(x, edge_index, edge_weight, W1, b1, W2, b2, W3, b3):
    N = x.shape[0]
    E = edge_weight.shape[0]
    Do = W1.shape[1]
    r = edge_index[0]
    c = edge_index[1]

    deg_parts = _make_deg(N, E)(c, edge_weight)
    dinv, xw1p = _tc_stage1(deg_parts, x, W1)

    acc_a, acc_b = _make_msg(N, Do, E // 2, 0, E // 2)(
        r, c, edge_weight, xw1p, xw1p)

    xw2p, xw3p = _tc_stage2(acc_a, acc_b, xw1p, dinv,
                            b1.reshape(1, -1), W2, W3)

    acc2, acc3 = _make_msg(N, Do, E, 0, 0)(r, c, edge_weight, xw2p, xw3p)

    mu, var = _tc_stage3(acc2, acc3, xw2p, xw3p, dinv,
                         b2.reshape(1, -1), b3.reshape(1, -1))
    return (mu, var)


# speedup vs baseline: 25.5654x; 1.0232x over previous
import functools

import jax
import jax.numpy as jnp
from jax import lax
from jax.experimental import pallas as pl
from jax.experimental.pallas import tpu as pltpu
from jax.experimental.pallas import tpu_sc as plsc

_NC = 2
_NS = 16


def _mesh():
    return plsc.VectorSubcoreMesh(core_axis_name="c", subcore_axis_name="s")


def _chunk_size(n, cap=128):
    for ch in range(cap - cap % 8, 7, -8):
        if n % ch == 0:
            return ch
    raise ValueError(f"no aligned chunk size for {n}")


def _make_deg(N, E):
    n_tiles = _NC * _NS
    ept = E // n_tiles
    assert ept * n_tiles == E and ept % 16 == 0 and N % 16 == 0

    @functools.partial(
        pl.kernel,
        out_type=jax.ShapeDtypeStruct((n_tiles, N), jnp.float32),
        mesh=_mesh(),
        compiler_params=pltpu.CompilerParams(needs_layout_passes=False),
        scratch_types=[
            pltpu.VMEM((ept,), jnp.int32),
            pltpu.VMEM((ept,), jnp.float32),
            pltpu.VMEM((N,), jnp.float32),
            pltpu.SemaphoreType.DMA,
        ],
    )
    def deg_kernel(c_hbm, w_hbm, out_hbm, cbuf, wbuf, deg_v, sem):
        tid = lax.axis_index("s") * _NC + lax.axis_index("c")
        base = tid * ept
        pltpu.async_copy(c_hbm.at[pl.ds(base, ept)], cbuf, sem)
        pltpu.async_copy(w_hbm.at[pl.ds(base, ept)], wbuf, sem)

        def zero_body(i, carry):
            deg_v[pl.ds(i * 16, 16)] = jnp.zeros((16,), jnp.float32)
            return carry

        lax.fori_loop(0, N // 16, zero_body, 0)
        pltpu.make_async_copy(c_hbm.at[pl.ds(base, ept)], cbuf, sem).wait()
        pltpu.make_async_copy(w_hbm.at[pl.ds(base, ept)], wbuf, sem).wait()

        def body(i, carry):
            idx = cbuf[pl.ds(i * 16, 16)]
            wv = wbuf[pl.ds(i * 16, 16)]
            plsc.addupdate_scatter(deg_v, [idx], wv)
            return carry

        lax.fori_loop(0, ept // 16, body, 0)
        pltpu.sync_copy(deg_v, out_hbm.at[tid])

    return deg_kernel


def _make_msg(N, Dj, cnt, base_a, base_b):
    ept = cnt // _NS
    assert ept * _NS == cnt
    ch = _chunk_size(ept)
    assert ch % 16 == 0
    nch = ept // ch
    assert nch >= 4
    rpt8 = (N // _NS) // 8 * 8
    extra = N - rpt8 * _NS
    zspan = rpt8 + extra
    zrows = 64
    assert zspan % zrows == 0 and extra % 8 == 0 and Dj % 16 == 0
    nz = zspan // zrows

    @functools.partial(
        pl.kernel,
        out_type=(jax.ShapeDtypeStruct((N, Dj), jnp.float32),
                  jax.ShapeDtypeStruct((N, Dj), jnp.float32)),
        mesh=_mesh(),
        scratch_types=[
            pltpu.VMEM((ch,), jnp.int32),
            pltpu.VMEM((ch,), jnp.int32),
            pltpu.VMEM((ch,), jnp.int32),
            pltpu.VMEM((ch,), jnp.int32),
            pltpu.VMEM((ch,), jnp.int32),
            pltpu.VMEM((ch,), jnp.int32),
            pltpu.VMEM((ch,), jnp.int32),
            pltpu.VMEM((ch,), jnp.int32),
            pltpu.VMEM((ch,), jnp.int32),
            pltpu.VMEM((ch,), jnp.float32),
            pltpu.VMEM((ch,), jnp.float32),
            pltpu.VMEM((ch,), jnp.float32),
            pltpu.VMEM((ch, Dj), jnp.float32),
            pltpu.VMEM((ch, Dj), jnp.float32),
            pltpu.VMEM((ch, Dj), jnp.float32),
            pltpu.VMEM((zrows, Dj), jnp.float32),
            pltpu.VMEM_SHARED((N, Dj), jnp.float32),
            pltpu.SemaphoreType.DMA,
            pltpu.SemaphoreType.DMA,
            pltpu.SemaphoreType.DMA,
        ],
    )
    def msg_kernel(r_hbm, c_hbm, w_hbm, ta_hbm, tb_hbm, oa_hbm, ob_hbm,
                   ridx0, ridx1, ridx2, cidx0, cidx1, cidx2,
                   scidx0, scidx1, scidx2, wv0, wv1, wv2,
                   rows0, rows1, rows2, zbuf, acc_sh, isem, gsem, ssem):
        cid = lax.axis_index("c")
        sid = lax.axis_index("s")
        ridx_bufs = (ridx0, ridx1, ridx2)
        cidx_bufs = (cidx0, cidx1, cidx2)
        scidx_bufs = (scidx0, scidx1, scidx2)
        wv_bufs = (wv0, wv1, wv2)
        rows_bufs = (rows0, rows1, rows2)

        def start_idx_at(base, g, b):
            off = base + g * ch
            pltpu.async_copy(r_hbm.at[pl.ds(off, ch)], ridx_bufs[b], isem)
            pltpu.async_copy(c_hbm.at[pl.ds(off, ch)], cidx_bufs[b], isem)
            pltpu.async_copy(w_hbm.at[pl.ds(off, ch)], wv_bufs[b], isem)

        @pl.when(cid == 0)
        def _():
            for g in range(3):
                start_idx_at(base_a + sid * ept, g, g)

        @pl.when(cid == 1)
        def _():
            for g in range(3):
                start_idx_at(base_b + sid * ept, g, g)

        def zb(i, carry):
            for k in range(Dj // 16):
                zbuf[i, pl.ds(k * 16, 16)] = jnp.zeros((16,), jnp.float32)
            return carry

        lax.fori_loop(0, zrows, zb, 0)
        row0 = pl.multiple_of(sid * rpt8, 8)
        for j in range(nz):
            pltpu.async_copy(zbuf, acc_sh.at[pl.ds(row0 + j * zrows, zrows)],
                             ssem)
        for j in range(nz):
            pltpu.make_async_copy(
                zbuf, acc_sh.at[pl.ds(row0 + j * zrows, zrows)], ssem).wait()

        def job(t_hbm, ebase):
            base = ebase + sid * ept

            def start_idx(g, b):
                start_idx_at(base, g, b)

            def wait_idx(g, b):
                off = base + g * ch
                pltpu.make_async_copy(
                    r_hbm.at[pl.ds(off, ch)], ridx_bufs[b], isem).wait()
                pltpu.make_async_copy(
                    c_hbm.at[pl.ds(off, ch)], cidx_bufs[b], isem).wait()
                pltpu.make_async_copy(
                    w_hbm.at[pl.ds(off, ch)], wv_bufs[b], isem).wait()

            def start_gather(b):
                pltpu.async_copy(t_hbm.at[ridx_bufs[b]], rows_bufs[b], gsem)

            def wait_gather(b):
                pltpu.make_async_copy(
                    t_hbm.at[ridx_bufs[b]], rows_bufs[b], gsem).wait()

            def scale(b):
                rows = rows_bufs[b]
                wv = wv_bufs[b]

                def sbody(q, c2):
                    w16 = wv[pl.ds(q * 16, 16)]
                    for j in range(16):
                        s = w16[j]
                        e = q * 16 + j
                        for k in range(Dj // 16):
                            sl = pl.ds(k * 16, 16)
                            rows[e, sl] = rows[e, sl] * s
                    return c2

                lax.fori_loop(0, ch // 16, sbody, 0)

            def issue_scatter(b):
                for q in range(ch // 16):
                    sl = pl.ds(q * 16, 16)
                    scidx_bufs[b][sl] = cidx_bufs[b][sl]
                pltpu.async_copy(rows_bufs[b], acc_sh.at[scidx_bufs[b]],
                                 ssem, add=True)

            def drain_scatter(b):
                pltpu.make_async_copy(
                    rows_bufs[b], acc_sh.at[scidx_bufs[b]], ssem).wait()

            def iteration(g, b, has_prev, has_n2, has_n3):
                b2 = (b + 2) % 3
                if has_n2:
                    wait_idx(g + 2, b2)
                    if has_prev:
                        drain_scatter(b2)
                    start_gather(b2)
                elif has_prev:
                    drain_scatter(b2)
                wait_gather(b)
                scale(b)
                issue_scatter(b)
                if has_n3:
                    start_idx(g + 3, b)

            wait_idx(0, 0)
            start_gather(0)
            wait_idx(1, 1)
            start_gather(1)
            plsc.subcore_barrier()
            iteration(0, 0, False, nch > 2, nch > 3)

            ntriples = (nch - 4) // 3

            def triple(p, carry):
                g = 1 + 3 * p
                iteration(g, 1, True, True, True)
                iteration(g + 1, 2, True, True, True)
                iteration(g + 2, 0, True, True, True)
                return carry

            lax.fori_loop(0, ntriples, triple, 0)

            for g in range(1 + 3 * ntriples, nch):
                iteration(g, g % 3, True, g + 2 < nch, g + 3 < nch)
            drain_scatter((nch - 1) % 3)

        @pl.when(cid == 0)
        def _():
            job(ta_hbm, base_a)

        @pl.when(cid == 1)
        def _():
            job(tb_hbm, base_b)

        plsc.subcore_barrier()

        def copy_out(o_hbm):
            pltpu.sync_copy(acc_sh.at[pl.ds(row0, rpt8)],
                            o_hbm.at[pl.ds(row0, rpt8)])

            @pl.when(sid == _NS - 1)
            def _():
                t0 = N - extra
                pltpu.sync_copy(acc_sh.at[pl.ds(t0, extra)],
                                o_hbm.at[pl.ds(t0, extra)])

        @pl.when(cid == 0)
        def _():
            copy_out(oa_hbm)

        @pl.when(cid == 1)
        def _():
            copy_out(ob_hbm)

    return msg_kernel


def _tc_stage1(deg_parts, x, W1):
    N = x.shape[0]
    Do = W1.shape[1]

    def body(dp_ref, x_ref, w_ref, dinv_ref, xwp_ref):
        deg = jnp.sum(dp_ref[...], axis=0) + 2.0
        dinv = lax.rsqrt(deg)
        dinv_ref[...] = dinv[:, None]
        xw = jnp.dot(x_ref[...], w_ref[...], preferred_element_type=jnp.float32)
        xwp_ref[...] = xw * dinv[:, None]

    return pl.pallas_call(
        body,
        out_shape=(jax.ShapeDtypeStruct((N, 1), jnp.float32),
                   jax.ShapeDtypeStruct((N, Do), jnp.float32)),
    )(deg_parts, x, W1)


def _tc_stage2(acc_lo, acc_hi, xw1p, dinv, b1, W2, W3):
    N, Do = xw1p.shape

    def body(lo_ref, hi_ref, xwp_ref, dv_ref, b_ref, w2_ref, w3_ref,
             o2_ref, o3_ref):
        acc = lo_ref[...] + hi_ref[...]
        dv = dv_ref[...]
        h = jnp.maximum(dv * (acc + 2.0 * xwp_ref[...]) + b_ref[...], 0.0)
        o2_ref[...] = dv * jnp.dot(h, w2_ref[...],
                                   preferred_element_type=jnp.float32)
        o3_ref[...] = dv * jnp.dot(h, w3_ref[...],
                                   preferred_element_type=jnp.float32)

    return pl.pallas_call(
        body,
        out_shape=(jax.ShapeDtypeStruct((N, Do), jnp.float32),
                   jax.ShapeDtypeStruct((N, Do), jnp.float32)),
    )(acc_lo, acc_hi, xw1p, dinv, b1, W2, W3)


def _tc_stage3(acc2, acc3, xw2p, xw3p, dinv, b2, b3):
    N, Do = xw2p.shape

    def body(a2_ref, a3_ref, x2_ref, x3_ref, dv_ref, b2_ref, b3_ref,
             mu_ref, var_ref):
        dv = dv_ref[...]
        mu_ref[...] = dv * (a2_ref[...] + 2.0 * x2_ref[...]) + b2_ref[...]
        var_ref[...] = dv * (a3_ref[...] + 2.0 * x3_ref[...]) + b3_ref[...]

    return pl.pallas_call(
        body,
        out_shape=(jax.ShapeDtypeStruct((N, Do), jnp.float32),
                   jax.ShapeDtypeStruct((N, Do), jnp.float32)),
    )(acc2, acc3, xw2p, xw3p, dinv, b2, b3)


def kernel(x, edge_index, edge_weight, W1, b1, W2, b2, W3, b3):
    N = x.shape[0]
    E = edge_weight.shape[0]
    Do = W1.shape[1]
    r = edge_index[0]
    c = edge_index[1]

    deg_parts = _make_deg(N, E)(c, edge_weight)
    dinv, xw1p = _tc_stage1(deg_parts, x, W1)

    acc_a, acc_b = _make_msg(N, Do, E // 2, 0, E // 2)(
        r, c, edge_weight, xw1p, xw1p)

    xw2p, xw3p = _tc_stage2(acc_a, acc_b, xw1p, dinv,
                            b1.reshape(1, -1), W2, W3)

    acc2, acc3 = _make_msg(N, Do, E, 0, 0)(r, c, edge_weight, xw2p, xw3p)

    mu, var = _tc_stage3(acc2, acc3, xw2p, xw3p, dinv,
                         b2.reshape(1, -1), b3.reshape(1, -1))
    return (mu, var)
